# baseline (speedup 1.0000x reference)
import functools

import jax
import jax.numpy as jnp
from jax import lax
from jax.experimental import pallas as pl
from jax.experimental.pallas import tpu as pltpu
from jax.experimental.pallas import tpu_sc as plsc

D_MODEL = 4096
NQ = 16
RANK = 64
BN = 1024

L = 16
SC_N = 8192
SC_CHUNKS = SC_N // L
SC_GROUPS = SC_CHUNKS // L
KSEL = 819
NOUT = 832


def _scores_body(x_ref, w_ref, q_ref, out_ref):
    x = x_ref[0]
    keys = lax.dot_general(
        x, w_ref[...], (((1,), (1,)), ((), ())),
        preferred_element_type=jnp.float32)
    st = lax.dot_general(
        q_ref[...], keys, (((1,), (1,)), ((), ())),
        preferred_element_type=jnp.float32)
    out_ref[0, 0] = jnp.max(st, axis=0)


def _token_scores(x, k_proj_w, queries, interpret=False):
    b, n, d = x.shape
    grid = (b, n // BN)
    out = pl.pallas_call(
        _scores_body,
        grid=grid,
        in_specs=[
            pl.BlockSpec((1, BN, d), lambda bi, i: (bi, i, 0)),
            pl.BlockSpec((RANK, d), lambda bi, i: (0, 0)),
            pl.BlockSpec((NQ, RANK), lambda bi, i: (0, 0)),
        ],
        out_specs=pl.BlockSpec((1, 1, BN), lambda bi, i: (bi, 0, i)),
        out_shape=jax.ShapeDtypeStruct((b, 1, n), jnp.float32),
        interpret=interpret,
    )(x, k_proj_w, queries)
    return out.reshape(b, n)


def _topk_body(scores_hbm, out_hbm, s_v, u_v, hist_v,
               cntg_v, cnte_v, baseg_v, basee_v, eqx_v, pos_v, out_v):
    cid = lax.axis_index("c")
    sid = lax.axis_index("s")
    wid = sid * 2 + cid

    @pl.when(wid < 4)
    def _work():
        lanes = lax.broadcasted_iota(jnp.int32, (L,), 0)
        pltpu.sync_copy(scores_hbm.at[wid], s_v)

        def conv(i, c):
            f = s_v[pl.ds(i * L, L)]
            ui = lax.bitcast_convert_type(f, jnp.uint32)
            neg = (ui >> jnp.uint32(31)) == jnp.uint32(1)
            u_v[pl.ds(i * L, L)] = jnp.where(
                neg, ~ui, ui | jnp.uint32(0x80000000))
            return c
        lax.fori_loop(0, SC_CHUNKS, conv, 0)

        def level(shift, himask, prefix, krem):
            def z(i, c):
                hist_v[pl.ds(i * L, L)] = jnp.zeros((L,), jnp.int32)
                return c
            lax.fori_loop(0, 256, z, 0)

            def h(i, c):
                u = u_v[pl.ds(i * L, L)]
                keep = (u & himask) == prefix
                bin_ = ((u >> jnp.uint32(shift))
                        & jnp.uint32(0xFF)).astype(jnp.int32)
                plsc.addupdate_scatter(
                    hist_v, [bin_ * L + lanes],
                    jnp.where(keep, 1, 0).astype(jnp.int32))
                return c
            lax.fori_loop(0, SC_CHUNKS, h, 0)

            def scan(j, carry):
                cum, bin_sel, krem_out, done = carry
                bidx = 255 - j
                t = jnp.sum(hist_v[pl.ds(bidx * L, L)])
                hit = jnp.logical_and(jnp.logical_not(done), cum + t >= krem)
                bin_sel = jnp.where(hit, bidx, bin_sel)
                krem_out = jnp.where(hit, krem - cum, krem_out)
                return cum + t, bin_sel, krem_out, jnp.logical_or(done, hit)
            _, bin_sel, krem2, _ = lax.fori_loop(
                0, 256, scan,
                (jnp.int32(0), jnp.int32(0), krem, jnp.bool_(False)))
            prefix2 = prefix | (bin_sel.astype(jnp.uint32)
                                << jnp.uint32(shift))
            return prefix2, krem2

        prefix = jnp.uint32(0)
        krem = jnp.int32(KSEL)
        prefix, krem = level(24, jnp.uint32(0x00000000), prefix, krem)
        prefix, krem = level(16, jnp.uint32(0xFF000000), prefix, krem)
        prefix, krem = level(8, jnp.uint32(0xFFFF0000), prefix, krem)
        prefix, krem = level(0, jnp.uint32(0xFFFFFF00), prefix, krem)
        u_k = prefix
        needed_eq = krem

        def cnt(i, c):
            u = u_v[pl.ds(i * L, L)]
            cntg_v[pl.ds(i * L, L)] = plsc.all_reduce_population_count(
                u > u_k)
            cnte_v[pl.ds(i * L, L)] = plsc.all_reduce_population_count(
                u == u_k)
            return c
        lax.fori_loop(0, SC_CHUNKS, cnt, 0)

        def scan_counts(cnt_ref, base_ref):
            lanes16 = lanes * L
            def g(gi, run):
                idx = gi * (L * L) + lanes16
                cvec = plsc.load_gather(cnt_ref, [idx])
                excl = plsc.cumsum(cvec) - cvec
                plsc.store_scatter(base_ref, [idx], excl + run)
                return run + jnp.sum(cvec)
            return lax.fori_loop(0, SC_GROUPS, g, jnp.int32(0))
        scan_counts(cntg_v, baseg_v)
        scan_counts(cnte_v, basee_v)

        def eqx(i, c):
            u = u_v[pl.ds(i * L, L)]
            eq_i = jnp.where(u == u_k, 1, 0).astype(jnp.int32)
            eqx_v[pl.ds(i * L, L)] = plsc.cumsum(eq_i) - eq_i
            return c
        lax.fori_loop(0, SC_CHUNKS, eqx, 0)

        def post(i, c):
            u = u_v[pl.ds(i * L, L)]
            bg = baseg_v[pl.ds(i * L, L)]
            be = basee_v[pl.ds(i * L, L)]
            eq_excl = eqx_v[pl.ds(i * L, L)]
            gt_i = jnp.where(u > u_k, 1, 0).astype(jnp.int32)
            eq_i = jnp.where(u == u_k, 1, 0).astype(jnp.int32)
            take_i = eq_i * jnp.where((be + eq_excl) < needed_eq, 1, 0)
            m_i = gt_i + take_i
            sel_excl = plsc.cumsum(m_i) - m_i
            be_cap = jnp.where(be < needed_eq, be, jnp.int32(0) + needed_eq)
            pos = bg + be_cap + sel_excl
            pos_v[pl.ds(i * L, L)] = jnp.where(m_i > 0, pos, NOUT + lanes)
            return c
        lax.fori_loop(0, SC_CHUNKS, post, 0)

        def comp(i, c):
            pos = pos_v[pl.ds(i * L, L)]
            plsc.store_scatter(out_v, [pos], i * L + lanes)
            return c
        lax.fori_loop(0, SC_CHUNKS, comp, 0)

        pltpu.sync_copy(out_v, out_hbm.at[wid])


@functools.partial(jax.jit, static_argnames=("interpret",))
def _topk_sc(scores, interpret=False):
    mesh = plsc.VectorSubcoreMesh(
        core_axis_name="c", subcore_axis_name="s",
        num_cores=2, num_subcores=16)
    f = pl.kernel(
        _topk_body,
        out_type=jax.ShapeDtypeStruct((4, NOUT + L), jnp.int32),
        mesh=mesh,
        scratch_types=[
            pltpu.VMEM((SC_N,), jnp.float32),
            pltpu.VMEM((SC_N,), jnp.uint32),
            pltpu.VMEM((256 * L,), jnp.int32),
            pltpu.VMEM((SC_N,), jnp.int32),
            pltpu.VMEM((SC_N,), jnp.int32),
            pltpu.VMEM((SC_N,), jnp.int32),
            pltpu.VMEM((SC_N,), jnp.int32),
            pltpu.VMEM((SC_N,), jnp.int32),
            pltpu.VMEM((SC_N,), jnp.int32),
            pltpu.VMEM((NOUT + L,), jnp.int32),
        ],
        compiler_params=pltpu.CompilerParams(needs_layout_passes=False),
        interpret=interpret,
    )
    return f(scores)


def kernel(x, k_proj_w, queries):
    b, n, d = x.shape
    token_scores = _token_scores(x, k_proj_w, queries)
    _, idx = lax.top_k(token_scores, KSEL)
    return jnp.sort(idx, axis=-1)
